# Initial kernel scaffold; baseline (speedup 1.0000x reference)
import jax
import jax.numpy as jnp
from jax.experimental import pallas as pl


def kernel(x_gt, x_pred, mask):
    raise NotImplementedError("write your pallas kernel here")


# speedup vs baseline: 2.7879x; 2.7879x over previous
import jax
import jax.numpy as jnp
from jax.experimental import pallas as pl


def _phase_a(xp_ref, xg_ref, m_ref, lp_ref, g2p_ref):
    t = pl.program_id(1)
    xp = xp_ref[0]
    xg = xg_ref[0]
    m = m_ref[0, 0]
    cross = jax.lax.dot_general(
        xp, xg, (((1,), (0,)), ((), ())),
        preferred_element_type=jnp.float32)
    p2 = jnp.sum(xp * xp, axis=1, keepdims=True)
    g2 = jnp.sum(xg * xg, axis=0, keepdims=True)
    d = jnp.maximum(p2 + g2 - 2.0 * cross, 0.0)

    pmin = jnp.min(d, axis=1, keepdims=True)
    lp_ref[0, 0] = jnp.where(m > 0.0, jnp.sqrt(pmin) * 100.0, jnp.inf)

    dm = d + jnp.where(m > 0.0, 0.0, jnp.inf)
    gmin = jnp.min(dm, axis=0, keepdims=True)

    @pl.when(t == 0)
    def _init():
        g2p_ref[0] = jnp.full(g2p_ref.shape[1:], jnp.inf, jnp.float32)

    g2p_ref[0] = jnp.minimum(g2p_ref[0], gmin)


def _phase_b(lp_ref, m_ref, g2p_ref, out_ref):
    lp = lp_ref[...]
    mv = m_ref[...]
    g2p = g2p_ref[...]

    n = jnp.sum(mv)
    idxf = jnp.float32(0.98) * (n - 1.0)
    low = jnp.floor(idxf)
    hw = idxf - low
    lw = 1.0 - hw
    low_i = jnp.clip(low, 0.0, n - 1.0).astype(jnp.int32)
    high_i = jnp.clip(low + 1.0, 0.0, n - 1.0).astype(jnp.int32)

    li = jax.lax.bitcast_convert_type(lp, jnp.int32)

    def body(i, prefix):
        bit = 30 - i
        one = jnp.int32(1)
        t_mid = prefix + (jax.lax.shift_left(one, bit) - 1)
        cnt = jnp.sum((li <= t_mid).astype(jnp.int32))
        return jnp.where(cnt >= low_i + 1, prefix,
                         prefix + jax.lax.shift_left(one, bit))

    s_low = jax.lax.fori_loop(0, 31, body, jnp.int32(0))
    s_low_f = jnp.max(jnp.where(li <= s_low, lp, -jnp.inf))
    cnt_le = jnp.sum((li <= s_low).astype(jnp.int32))
    nxt = jnp.min(jnp.where(li > s_low, lp, jnp.inf))
    s_high_f = jnp.where(cnt_le >= high_i + 1, s_low_f, nxt)
    q = s_low_f * lw + s_high_f * hw

    keep = lp <= q
    lp_mean = jnp.sum(jnp.where(keep, lp, 0.0)) / jnp.sum(
        keep.astype(jnp.float32))
    g_mean = jnp.mean(jnp.sqrt(g2p) * 100.0)
    out_ref[...] = jnp.broadcast_to(lp_mean + g_mean, (1, 1))


def kernel(x_gt, x_pred, mask):
    B, V1, _ = x_gt.shape
    V2 = x_pred.shape[1]
    TP = 512
    NT = V2 // TP

    xg_t = jnp.swapaxes(x_gt, 1, 2)
    m4 = mask.astype(jnp.float32).reshape(B, NT, TP, 1)

    lp4, g2p = pl.pallas_call(
        _phase_a,
        grid=(B, NT),
        in_specs=[
            pl.BlockSpec((1, TP, 3), lambda b, t: (b, t, 0)),
            pl.BlockSpec((1, 3, V1), lambda b, t: (b, 0, 0)),
            pl.BlockSpec((1, 1, TP, 1), lambda b, t: (b, t, 0, 0)),
        ],
        out_specs=[
            pl.BlockSpec((1, 1, TP, 1), lambda b, t: (b, t, 0, 0)),
            pl.BlockSpec((1, 1, V1), lambda b, t: (b, 0, 0)),
        ],
        out_shape=[
            jax.ShapeDtypeStruct((B, NT, TP, 1), jnp.float32),
            jax.ShapeDtypeStruct((B, 1, V1), jnp.float32),
        ],
    )(x_pred, xg_t, m4)

    out = pl.pallas_call(
        _phase_b,
        out_shape=jax.ShapeDtypeStruct((1, 1), jnp.float32),
    )(lp4, m4, g2p)
    return out.reshape(())


# speedup vs baseline: 3.2045x; 1.1494x over previous
import jax
import jax.numpy as jnp
from jax.experimental import pallas as pl
from jax.experimental.pallas import tpu as pltpu


def _body(B, NT, TP, V1, xp_ref, xg_ref, m_ref, out_ref, lp_s, g2p_s):
    i = pl.program_id(0)

    @pl.when(i < B * NT)
    def _phase_a():
        b = i // NT
        t = i % NT
        xp = xp_ref[0]
        xg = xg_ref[0]
        m = m_ref[b, t]
        p2 = jnp.sum(xp * xp, axis=1, keepdims=True)
        pm = p2 + jnp.where(m > 0.0, 0.0, jnp.inf)
        g2 = jnp.sum(xg * xg, axis=0, keepdims=True)
        cross = jax.lax.dot_general(
            xp, xg, (((1,), (0,)), ((), ())),
            preferred_element_type=jnp.float32)
        e = g2 - 2.0 * cross
        pmin = jnp.min(e, axis=1, keepdims=True) + p2
        lp_s[b, t] = jnp.where(
            m > 0.0, jnp.sqrt(jnp.maximum(pmin, 0.0)) * 100.0, jnp.inf)
        gmin = jnp.min(e + pm, axis=0, keepdims=True)

        @pl.when(t == 0)
        def _first():
            g2p_s[b] = gmin

        @pl.when(t > 0)
        def _rest():
            g2p_s[b] = jnp.minimum(g2p_s[b], gmin)

    @pl.when(i == B * NT)
    def _phase_b():
        lp = lp_s[...]
        mv = m_ref[...]
        n = jnp.sum(mv)
        idxf = jnp.float32(0.98) * (n - 1.0)
        low = jnp.floor(idxf)
        hw = idxf - low
        lw = 1.0 - hw
        low_i = jnp.clip(low, 0.0, n - 1.0).astype(jnp.int32)
        high_i = jnp.clip(low + 1.0, 0.0, n - 1.0).astype(jnp.int32)

        li = jax.lax.bitcast_convert_type(lp, jnp.int32)

        def body(k, prefix):
            bit = 30 - k
            one = jnp.int32(1)
            t_mid = prefix + (jax.lax.shift_left(one, bit) - 1)
            cnt = jnp.sum((li <= t_mid).astype(jnp.int32))
            return jnp.where(cnt >= low_i + 1, prefix,
                             prefix + jax.lax.shift_left(one, bit))

        s_low = jax.lax.fori_loop(0, 31, body, jnp.int32(0))
        s_low_f = jnp.max(jnp.where(li <= s_low, lp, -jnp.inf))
        cnt_le = jnp.sum((li <= s_low).astype(jnp.int32))
        nxt = jnp.min(jnp.where(li > s_low, lp, jnp.inf))
        s_high_f = jnp.where(cnt_le >= high_i + 1, s_low_f, nxt)
        q = s_low_f * lw + s_high_f * hw

        keep = lp <= q
        lp_mean = jnp.sum(jnp.where(keep, lp, 0.0)) / jnp.sum(
            keep.astype(jnp.float32))
        gl = jnp.sqrt(jnp.maximum(g2p_s[...], 0.0)) * 100.0
        out_ref[...] = jnp.broadcast_to(lp_mean + jnp.mean(gl), (1, 1))


def kernel(x_gt, x_pred, mask):
    B, V1, _ = x_gt.shape
    V2 = x_pred.shape[1]
    TP = 512
    NT = V2 // TP

    xg_t = jnp.swapaxes(x_gt, 1, 2)
    m4 = mask.astype(jnp.float32).reshape(B, NT, TP, 1)

    def fused(xp_ref, xg_ref, m_ref, out_ref, lp_s, g2p_s):
        _body(B, NT, TP, V1, xp_ref, xg_ref, m_ref, out_ref, lp_s, g2p_s)

    out = pl.pallas_call(
        fused,
        grid=(B * NT + 1,),
        in_specs=[
            pl.BlockSpec((1, TP, 3),
                         lambda i: (jnp.minimum(i // NT, B - 1), i % NT, 0)),
            pl.BlockSpec((1, 3, V1),
                         lambda i: (jnp.minimum(i // NT, B - 1), 0, 0)),
            pl.BlockSpec((B, NT, TP, 1), lambda i: (0, 0, 0, 0)),
        ],
        out_specs=pl.BlockSpec((1, 1), lambda i: (0, 0)),
        out_shape=jax.ShapeDtypeStruct((1, 1), jnp.float32),
        scratch_shapes=[
            pltpu.VMEM((B, NT, TP, 1), jnp.float32),
            pltpu.VMEM((B, 1, V1), jnp.float32),
        ],
    )(x_pred, xg_t, m4)
    return out.reshape(())


# speedup vs baseline: 3.7562x; 1.1722x over previous
import jax
import jax.numpy as jnp
from jax.experimental import pallas as pl
from jax.experimental.pallas import tpu as pltpu


def _body(B, NT, TP, V1, xp_ref, xpt_ref, xg_ref, xgt_ref, mrow_ref,
          mcol_ref, mfull_ref, out_ref, lp_s, g2p_s, g2r_s):
    i = pl.program_id(0)

    @pl.when(i < B * NT)
    def _phase_a():
        b = i // NT
        t = i % NT
        xp = xp_ref[0]
        xpt = xpt_ref[0]
        xg = xg_ref[0]
        xgt = xgt_ref[0]

        @pl.when(t == 0)
        def _per_batch():
            g2r_s[b] = jnp.sum(xgt * xgt, axis=0, keepdims=True)

        g2c = jnp.sum(xg * xg, axis=1, keepdims=True)
        crosst = jax.lax.dot_general(
            xg, xpt, (((1,), (0,)), ((), ())),
            preferred_element_type=jnp.float32)
        et = g2c - 2.0 * crosst
        lpmin = jnp.min(et, axis=0, keepdims=True)
        p2r = jnp.sum(xpt * xpt, axis=0, keepdims=True)
        mrow = mrow_ref[0]
        lp_s[i] = jnp.where(
            mrow > 0.0,
            jnp.sqrt(jnp.maximum(lpmin + p2r, 0.0)) * 100.0, jnp.inf)

        p2c = jnp.sum(xp * xp, axis=1, keepdims=True)
        pmc = p2c + jnp.where(mcol_ref[0, 0] > 0.0, 0.0, jnp.inf)
        cross = jax.lax.dot_general(
            xp, xgt, (((1,), (0,)), ((), ())),
            preferred_element_type=jnp.float32)
        f = pmc - 2.0 * cross
        gmin = jnp.min(f, axis=0, keepdims=True)

        @pl.when(t == 0)
        def _first():
            g2p_s[b] = gmin

        @pl.when(t > 0)
        def _rest():
            g2p_s[b] = jnp.minimum(g2p_s[b], gmin)

    @pl.when(i == B * NT)
    def _phase_b():
        lp = lp_s[...]
        mv = mfull_ref[...]
        n = jnp.sum(mv)
        idxf = jnp.float32(0.98) * (n - 1.0)
        low = jnp.floor(idxf)
        hw = idxf - low
        lw = 1.0 - hw
        low_i = jnp.clip(low, 0.0, n - 1.0).astype(jnp.int32)
        high_i = jnp.clip(low + 1.0, 0.0, n - 1.0).astype(jnp.int32)

        li = jax.lax.bitcast_convert_type(lp, jnp.int32)

        def body(k, prefix):
            bit = 30 - k
            one = jnp.int32(1)
            t_mid = prefix + (jax.lax.shift_left(one, bit) - 1)
            cnt = jnp.sum((li <= t_mid).astype(jnp.int32))
            return jnp.where(cnt >= low_i + 1, prefix,
                             prefix + jax.lax.shift_left(one, bit))

        s_low = jax.lax.fori_loop(0, 31, body, jnp.int32(0))
        s_low_f = jnp.max(jnp.where(li <= s_low, lp, -jnp.inf))
        cnt_le = jnp.sum((li <= s_low).astype(jnp.int32))
        nxt = jnp.min(jnp.where(li > s_low, lp, jnp.inf))
        s_high_f = jnp.where(cnt_le >= high_i + 1, s_low_f, nxt)
        q = s_low_f * lw + s_high_f * hw

        keep = lp <= q
        lp_mean = jnp.sum(jnp.where(keep, lp, 0.0)) / jnp.sum(
            keep.astype(jnp.float32))
        gl = jnp.sqrt(jnp.maximum(g2p_s[...] + g2r_s[...], 0.0)) * 100.0
        out_ref[...] = jnp.broadcast_to(lp_mean + jnp.mean(gl), (1, 1))


def kernel(x_gt, x_pred, mask):
    B, V1, _ = x_gt.shape
    V2 = x_pred.shape[1]
    TP = 512
    NT = V2 // TP

    xp_t = jnp.swapaxes(x_pred, 1, 2)
    xg_t = jnp.swapaxes(x_gt, 1, 2)
    mf = mask.astype(jnp.float32)
    m_row = mf.reshape(B * NT, 1, TP)
    m_col = mf.reshape(B, NT, TP, 1)

    def fused(*refs):
        _body(B, NT, TP, V1, *refs)

    out = pl.pallas_call(
        fused,
        grid=(B * NT + 1,),
        in_specs=[
            pl.BlockSpec((1, TP, 3),
                         lambda i: (jnp.minimum(i // NT, B - 1), i % NT, 0)),
            pl.BlockSpec((1, 3, TP),
                         lambda i: (jnp.minimum(i // NT, B - 1), 0, i % NT)),
            pl.BlockSpec((1, V1, 3),
                         lambda i: (jnp.minimum(i // NT, B - 1), 0, 0)),
            pl.BlockSpec((1, 3, V1),
                         lambda i: (jnp.minimum(i // NT, B - 1), 0, 0)),
            pl.BlockSpec((1, 1, TP),
                         lambda i: (jnp.minimum(i, B * NT - 1), 0, 0)),
            pl.BlockSpec((1, 1, TP, 1),
                         lambda i: (jnp.minimum(i // NT, B - 1), i % NT,
                                    0, 0)),
            pl.BlockSpec((B * NT, 1, TP), lambda i: (0, 0, 0)),
        ],
        out_specs=pl.BlockSpec((1, 1), lambda i: (0, 0)),
        out_shape=jax.ShapeDtypeStruct((1, 1), jnp.float32),
        scratch_shapes=[
            pltpu.VMEM((B * NT, 1, TP), jnp.float32),
            pltpu.VMEM((B, 1, V1), jnp.float32),
            pltpu.VMEM((B, 1, V1), jnp.float32),
        ],
    )(x_pred, xp_t, x_gt, xg_t, m_row, m_col, m_row)
    return out.reshape(())
